# Initial kernel scaffold; baseline (speedup 1.0000x reference)
import jax
import jax.numpy as jnp
from jax.experimental import pallas as pl


def kernel(x, edge_index, W, att_src, att_dst, bias):
    raise NotImplementedError("write your pallas kernel here")


# speedup vs baseline: 27.6005x; 27.6005x over previous
import functools

import jax
import jax.numpy as jnp
from jax import lax
from jax.experimental import pallas as pl
from jax.experimental.pallas import tpu as pltpu
from jax.experimental.pallas import tpu_sc as plsc

N = 10000
E = 320000
C = 128
NEG_SLOPE = 0.2

NC = 2
NS = 16
NW = NC * NS
EW = E // NW
K = 80
NCHUNK = EW // K
NA = 10112
RPT = NA // NS

ROW_BLK = 1000
GRID = N // ROW_BLK


def _prep_body(x_ref, w_ref, as_ref, ad_ref,
               h_ref, asrc_ref, adst_ref, m_ref, mx_s):
    i = pl.program_id(0)
    h = jnp.dot(x_ref[...], w_ref[...], preferred_element_type=jnp.float32)
    h_ref[...] = h
    a_s = jnp.sum(h * as_ref[...], axis=1)
    a_d = jnp.sum(h * ad_ref[...], axis=1)
    asrc_ref[...] = a_s.reshape(1, 1, ROW_BLK)
    adst_ref[...] = a_d.reshape(1, 1, ROW_BLK)
    bs = jnp.max(a_s)
    bd = jnp.max(a_d)

    @pl.when(i == 0)
    def _init():
        mx_s[0] = bs
        mx_s[1] = bd

    @pl.when(i > 0)
    def _acc():
        mx_s[0] = jnp.maximum(mx_s[0], bs)
        mx_s[1] = jnp.maximum(mx_s[1], bd)

    mm = mx_s[0] + mx_s[1]
    mm = jnp.where(mm >= 0.0, mm, mm * NEG_SLOPE)
    m_ref[...] = jnp.full((C,), mm, dtype=jnp.float32)


def _prep(x, W, att_s, att_d):
    return pl.pallas_call(
        _prep_body,
        grid=(GRID,),
        in_specs=[
            pl.BlockSpec((ROW_BLK, C), lambda i: (i, 0)),
            pl.BlockSpec((C, C), lambda i: (0, 0)),
            pl.BlockSpec((1, C), lambda i: (0, 0)),
            pl.BlockSpec((1, C), lambda i: (0, 0)),
        ],
        out_specs=[
            pl.BlockSpec((ROW_BLK, C), lambda i: (i, 0)),
            pl.BlockSpec((1, 1, ROW_BLK), lambda i: (i, 0, 0)),
            pl.BlockSpec((1, 1, ROW_BLK), lambda i: (i, 0, 0)),
            pl.BlockSpec((C,), lambda i: (0,)),
        ],
        out_shape=[
            jax.ShapeDtypeStruct((N, C), jnp.float32),
            jax.ShapeDtypeStruct((GRID, 1, ROW_BLK), jnp.float32),
            jax.ShapeDtypeStruct((GRID, 1, ROW_BLK), jnp.float32),
            jax.ShapeDtypeStruct((C,), jnp.float32),
        ],
        scratch_shapes=[pltpu.SMEM((2,), jnp.float32)],
    )(x, W, att_s, att_d)


def _sc_body(h_hbm, asrc_hbm, adst_hbm, m_hbm, src_hbm, dst_hbm,
             zr_hbm, zv_hbm,
             outp_hbm, denp_hbm,
             asrc_t, adst_t, m_v, denom_p,
             rows_v, sidx, didx, acc, gsem):
    cid = lax.axis_index("c")
    sid = lax.axis_index("s")
    wid = sid * NC + cid
    ebase = wid * EW

    pltpu.sync_copy(asrc_hbm, asrc_t)
    pltpu.sync_copy(adst_hbm, adst_t)
    pltpu.sync_copy(m_hbm.at[pl.ds(0, 16)], m_v)
    pltpu.sync_copy(zv_hbm, denom_p)
    pltpu.sync_copy(zr_hbm, acc.at[pl.ds(sid * RPT, RPT)])

    mv = m_v[...]

    plsc.subcore_barrier()

    def pass2(ci, carry):
        pltpu.sync_copy(src_hbm.at[pl.ds(ebase + ci * K, K)], sidx)
        pltpu.sync_copy(dst_hbm.at[pl.ds(ebase + ci * K, K)], didx)
        pltpu.async_copy(h_hbm.at[sidx], rows_v, gsem).wait()

        for g in range(K // 16):
            sv = sidx[pl.ds(g * 16, 16)]
            dv = didx[pl.ds(g * 16, 16)]
            a1 = plsc.load_gather(asrc_t, [sv])
            a2 = plsc.load_gather(adst_t, [dv])
            al = a1 + a2
            al = jnp.where(al >= 0.0, al, al * NEG_SLOPE)
            exv = jnp.exp(al - mv)
            plsc.addupdate_scatter(denom_p, [dv], exv)
            for e2 in range(16):
                s = exv[e2]
                row = g * 16 + e2
                for j in range(C // 16):
                    rows_v[row, pl.ds(j * 16, 16)] = (
                        rows_v[row, pl.ds(j * 16, 16)] * s)

        pltpu.sync_copy(rows_v, acc.at[didx], add=True)
        return carry

    lax.fori_loop(0, NCHUNK, pass2, 0)

    plsc.subcore_barrier()

    pltpu.sync_copy(acc.at[pl.ds(sid * RPT, RPT)],
                    outp_hbm.at[cid, pl.ds(sid * RPT, RPT)])
    pltpu.sync_copy(denom_p, denp_hbm.at[wid])


def _edge_pass(h, a_src, a_dst, m, src, dst, zr, zv):
    mesh = plsc.VectorSubcoreMesh(core_axis_name="c", subcore_axis_name="s")
    fn = functools.partial(
        pl.kernel,
        mesh=mesh,
        compiler_params=pltpu.CompilerParams(needs_layout_passes=False),
        out_type=[
            jax.ShapeDtypeStruct((NC, NA, C), jnp.float32),
            jax.ShapeDtypeStruct((NW, N), jnp.float32),
        ],
        scratch_types=[
            pltpu.VMEM((N,), jnp.float32),
            pltpu.VMEM((N,), jnp.float32),
            pltpu.VMEM((16,), jnp.float32),
            pltpu.VMEM((N,), jnp.float32),
            pltpu.VMEM((K, C), jnp.float32),
            pltpu.VMEM((K,), jnp.int32),
            pltpu.VMEM((K,), jnp.int32),
            pltpu.VMEM_SHARED((NA, C), jnp.float32),
            pltpu.SemaphoreType.DMA,
        ],
    )(_sc_body)
    return fn(h, a_src, a_dst, m, src, dst, zr, zv)


def _finish_body(outp_ref, denp_ref, h_ref, asrc_ref, adst_ref, m_ref, b_ref,
                 o_ref):
    s = outp_ref[0] + outp_ref[1]
    d = jnp.sum(denp_ref[:, 0, 0, :], axis=0)
    mm = jnp.max(m_ref[...])
    al = asrc_ref[0, 0] + adst_ref[0, 0]
    al = jnp.where(al >= 0.0, al, al * NEG_SLOPE)
    exl = jnp.exp(al - mm)
    s = s + exl[:, None] * h_ref[...]
    d = d + exl
    o = s / (d + 1e-16)[:, None] + b_ref[...][None, :]
    mx = jnp.max(o, axis=1, keepdims=True)
    lo = o - mx
    o_ref[...] = lo - jnp.log(jnp.sum(jnp.exp(lo), axis=1, keepdims=True))


def _finish(outp, denp, h, a_src, a_dst, m, bias):
    return pl.pallas_call(
        _finish_body,
        grid=(GRID,),
        in_specs=[
            pl.BlockSpec((NC, ROW_BLK, C), lambda i: (0, i, 0)),
            pl.BlockSpec((NW, 1, 1, ROW_BLK), lambda i: (0, i, 0, 0)),
            pl.BlockSpec((ROW_BLK, C), lambda i: (i, 0)),
            pl.BlockSpec((1, 1, ROW_BLK), lambda i: (i, 0, 0)),
            pl.BlockSpec((1, 1, ROW_BLK), lambda i: (i, 0, 0)),
            pl.BlockSpec((C,), lambda i: (0,)),
            pl.BlockSpec((C,), lambda i: (0,)),
        ],
        out_specs=pl.BlockSpec((ROW_BLK, C), lambda i: (i, 0)),
        out_shape=jax.ShapeDtypeStruct((N, C), jnp.float32),
    )(outp, denp, h, a_src, a_dst, m, bias)


def kernel(x, edge_index, W, att_src, att_dst, bias):
    att_s = att_src.reshape(1, C)
    att_d = att_dst.reshape(1, C)
    h, a_src3, a_dst3, m = _prep(x, W, att_s, att_d)
    a_src = a_src3.reshape(N)
    a_dst = a_dst3.reshape(N)
    src = edge_index[0]
    dst = edge_index[1]
    zr = jnp.zeros((RPT, C), jnp.float32)
    zv = jnp.zeros((N,), jnp.float32)
    outp, denp = _edge_pass(h, a_src, a_dst, m, src, dst, zr, zv)
    denp_r = denp.reshape(NW, GRID, 1, ROW_BLK)
    return _finish(outp[:, :N, :], denp_r, h, a_src3, a_dst3, m, bias)


# speedup vs baseline: 33.1546x; 1.2012x over previous
import functools

import jax
import jax.numpy as jnp
from jax import lax
from jax.experimental import pallas as pl
from jax.experimental.pallas import tpu as pltpu
from jax.experimental.pallas import tpu_sc as plsc

N = 10000
E = 320000
C = 128
NEG_SLOPE = 0.2

NC = 2
NS = 16
NW = NC * NS
EW = E // NW
K = 80
NCHUNK = EW // K
NA = 10112
RPT = NA // NS

ROW_BLK = 1000
GRID = N // ROW_BLK


def _prep_body(x_ref, w_ref, as_ref, ad_ref,
               h_ref, asrc_ref, adst_ref, m_ref, mx_s):
    i = pl.program_id(0)
    h = jnp.dot(x_ref[...], w_ref[...], preferred_element_type=jnp.float32)
    h_ref[...] = h
    a_s = jnp.sum(h * as_ref[...], axis=1)
    a_d = jnp.sum(h * ad_ref[...], axis=1)
    asrc_ref[...] = a_s.reshape(1, 1, ROW_BLK)
    adst_ref[...] = a_d.reshape(1, 1, ROW_BLK)
    bs = jnp.max(a_s)
    bd = jnp.max(a_d)

    @pl.when(i == 0)
    def _init():
        mx_s[0] = bs
        mx_s[1] = bd

    @pl.when(i > 0)
    def _acc():
        mx_s[0] = jnp.maximum(mx_s[0], bs)
        mx_s[1] = jnp.maximum(mx_s[1], bd)

    mm = mx_s[0] + mx_s[1]
    mm = jnp.where(mm >= 0.0, mm, mm * NEG_SLOPE)
    m_ref[...] = jnp.full((C,), mm, dtype=jnp.float32)


def _prep(x, W, att_s, att_d):
    return pl.pallas_call(
        _prep_body,
        grid=(GRID,),
        in_specs=[
            pl.BlockSpec((ROW_BLK, C), lambda i: (i, 0)),
            pl.BlockSpec((C, C), lambda i: (0, 0)),
            pl.BlockSpec((1, C), lambda i: (0, 0)),
            pl.BlockSpec((1, C), lambda i: (0, 0)),
        ],
        out_specs=[
            pl.BlockSpec((ROW_BLK, C), lambda i: (i, 0)),
            pl.BlockSpec((1, 1, ROW_BLK), lambda i: (i, 0, 0)),
            pl.BlockSpec((1, 1, ROW_BLK), lambda i: (i, 0, 0)),
            pl.BlockSpec((C,), lambda i: (0,)),
        ],
        out_shape=[
            jax.ShapeDtypeStruct((N, C), jnp.float32),
            jax.ShapeDtypeStruct((GRID, 1, ROW_BLK), jnp.float32),
            jax.ShapeDtypeStruct((GRID, 1, ROW_BLK), jnp.float32),
            jax.ShapeDtypeStruct((C,), jnp.float32),
        ],
        scratch_shapes=[pltpu.SMEM((2,), jnp.float32)],
    )(x, W, att_s, att_d)


def _sc_body(h_hbm, asrc_hbm, adst_hbm, m_hbm, src_hbm, dst_hbm,
             zr_hbm, zv_hbm,
             outp_hbm, denp_hbm,
             asrc_t, adst_t, m_v, ex_c,
             rows_a, rows_b, sidx_a, sidx_b, didx,
             acc, den_sh, sem_a, sem_b):
    cid = lax.axis_index("c")
    sid = lax.axis_index("s")
    ebase = (sid * NC + cid) * EW

    pltpu.sync_copy(asrc_hbm, asrc_t)
    pltpu.sync_copy(adst_hbm, adst_t)
    pltpu.sync_copy(m_hbm.at[pl.ds(0, 16)], m_v)
    pltpu.sync_copy(zr_hbm, acc.at[pl.ds(sid * RPT, RPT)])

    @pl.when(sid == 0)
    def _zero_den():
        pltpu.sync_copy(zv_hbm, den_sh)

    mv = m_v[...]

    plsc.subcore_barrier()

    def fire(ci, sbuf, rbuf, sem):
        pltpu.sync_copy(src_hbm.at[pl.ds(ebase + ci * K, K)], sbuf)
        pltpu.async_copy(h_hbm.at[sbuf], rbuf, sem)

    def wait(sbuf, rbuf, sem):
        pltpu.make_async_copy(h_hbm.at[sbuf], rbuf, sem).wait()

    def process(ci, sbuf, rbuf):
        pltpu.sync_copy(dst_hbm.at[pl.ds(ebase + ci * K, K)], didx)
        for g in range(K // 16):
            sv = sbuf[pl.ds(g * 16, 16)]
            dv = didx[pl.ds(g * 16, 16)]
            a1 = plsc.load_gather(asrc_t, [sv])
            a2 = plsc.load_gather(adst_t, [dv])
            al = a1 + a2
            al = jnp.where(al >= 0.0, al, al * NEG_SLOPE)
            exv = jnp.exp(al - mv)
            ex_c[pl.ds(g * 16, 16)] = exv
            for e2 in range(16):
                s = exv[e2]
                row = g * 16 + e2
                for j in range(C // 16):
                    rbuf[row, pl.ds(j * 16, 16)] = (
                        rbuf[row, pl.ds(j * 16, 16)] * s)
        pltpu.sync_copy(ex_c, den_sh.at[didx], add=True)
        pltpu.sync_copy(rbuf, acc.at[didx], add=True)

    fire(0, sidx_a, rows_a, sem_a)

    def pair(i, carry):
        ca = 2 * i
        wait(sidx_a, rows_a, sem_a)
        fire(ca + 1, sidx_b, rows_b, sem_b)
        process(ca, sidx_a, rows_a)
        wait(sidx_b, rows_b, sem_b)
        fire(ca + 2, sidx_a, rows_a, sem_a)
        process(ca + 1, sidx_b, rows_b)
        return carry

    lax.fori_loop(0, (NCHUNK - 1) // 2, pair, 0)

    wait(sidx_a, rows_a, sem_a)
    process(NCHUNK - 1, sidx_a, rows_a)

    plsc.subcore_barrier()

    pltpu.sync_copy(acc.at[pl.ds(sid * RPT, RPT)],
                    outp_hbm.at[cid, pl.ds(sid * RPT, RPT)])

    @pl.when(sid == 0)
    def _den_out():
        pltpu.sync_copy(den_sh, asrc_t)
        pltpu.sync_copy(asrc_t, denp_hbm.at[cid])


def _edge_pass(h, a_src, a_dst, m, src, dst, zr, zv):
    mesh = plsc.VectorSubcoreMesh(core_axis_name="c", subcore_axis_name="s")
    fn = functools.partial(
        pl.kernel,
        mesh=mesh,
        compiler_params=pltpu.CompilerParams(needs_layout_passes=False),
        out_type=[
            jax.ShapeDtypeStruct((NC, NA, C), jnp.float32),
            jax.ShapeDtypeStruct((NC, N), jnp.float32),
        ],
        scratch_types=[
            pltpu.VMEM((N,), jnp.float32),
            pltpu.VMEM((N,), jnp.float32),
            pltpu.VMEM((16,), jnp.float32),
            pltpu.VMEM((K,), jnp.float32),
            pltpu.VMEM((K, C), jnp.float32),
            pltpu.VMEM((K, C), jnp.float32),
            pltpu.VMEM((K,), jnp.int32),
            pltpu.VMEM((K,), jnp.int32),
            pltpu.VMEM((K,), jnp.int32),
            pltpu.VMEM_SHARED((NA, C), jnp.float32),
            pltpu.VMEM_SHARED((N,), jnp.float32),
            pltpu.SemaphoreType.DMA,
            pltpu.SemaphoreType.DMA,
        ],
    )(_sc_body)
    return fn(h, a_src, a_dst, m, src, dst, zr, zv)


def _finish_body(outp_ref, denp_ref, h_ref, asrc_ref, adst_ref, m_ref, b_ref,
                 o_ref):
    s = outp_ref[0] + outp_ref[1]
    d = jnp.sum(denp_ref[:, 0, 0, :], axis=0)
    mm = jnp.max(m_ref[...])
    al = asrc_ref[0, 0] + adst_ref[0, 0]
    al = jnp.where(al >= 0.0, al, al * NEG_SLOPE)
    exl = jnp.exp(al - mm)
    s = s + exl[:, None] * h_ref[...]
    d = d + exl
    o = s / (d + 1e-16)[:, None] + b_ref[...][None, :]
    mx = jnp.max(o, axis=1, keepdims=True)
    lo = o - mx
    o_ref[...] = lo - jnp.log(jnp.sum(jnp.exp(lo), axis=1, keepdims=True))


def _finish(outp, denp, h, a_src, a_dst, m, bias):
    return pl.pallas_call(
        _finish_body,
        grid=(GRID,),
        in_specs=[
            pl.BlockSpec((NC, ROW_BLK, C), lambda i: (0, i, 0)),
            pl.BlockSpec((NC, 1, 1, ROW_BLK), lambda i: (0, i, 0, 0)),
            pl.BlockSpec((ROW_BLK, C), lambda i: (i, 0)),
            pl.BlockSpec((1, 1, ROW_BLK), lambda i: (i, 0, 0)),
            pl.BlockSpec((1, 1, ROW_BLK), lambda i: (i, 0, 0)),
            pl.BlockSpec((C,), lambda i: (0,)),
            pl.BlockSpec((C,), lambda i: (0,)),
        ],
        out_specs=pl.BlockSpec((ROW_BLK, C), lambda i: (i, 0)),
        out_shape=jax.ShapeDtypeStruct((N, C), jnp.float32),
    )(outp, denp, h, a_src, a_dst, m, bias)


def kernel(x, edge_index, W, att_src, att_dst, bias):
    att_s = att_src.reshape(1, C)
    att_d = att_dst.reshape(1, C)
    h, a_src3, a_dst3, m = _prep(x, W, att_s, att_d)
    a_src = a_src3.reshape(N)
    a_dst = a_dst3.reshape(N)
    src = edge_index[0]
    dst = edge_index[1]
    zr = jnp.zeros((RPT, C), jnp.float32)
    zv = jnp.zeros((N,), jnp.float32)
    outp, denp = _edge_pass(h, a_src, a_dst, m, src, dst, zr, zv)
    denp_r = denp.reshape(NC, GRID, 1, ROW_BLK)
    return _finish(outp[:, :N, :], denp_r, h, a_src3, a_dst3, m, bias)


# speedup vs baseline: 41.1128x; 1.2400x over previous
import functools

import jax
import jax.numpy as jnp
from jax import lax
from jax.experimental import pallas as pl
from jax.experimental.pallas import tpu as pltpu
from jax.experimental.pallas import tpu_sc as plsc

N = 10000
E = 320000
C = 128
NEG_SLOPE = 0.2

NC = 2
NS = 16
NW = NC * NS
EW = E // NW
K = 80
NCHUNK = EW // K
NA = 10112
RPT = NA // NS

ROW_BLK = 1000
GRID = N // ROW_BLK


def _prep_body(x_ref, w_ref, as_ref, ad_ref,
               h_ref, asrc_ref, adst_ref, m_ref, mx_s):
    i = pl.program_id(0)
    h = jnp.dot(x_ref[...], w_ref[...], preferred_element_type=jnp.float32)
    h_ref[...] = h
    a_s = jnp.sum(h * as_ref[...], axis=1)
    a_d = jnp.sum(h * ad_ref[...], axis=1)
    asrc_ref[...] = a_s.reshape(1, 1, ROW_BLK)
    adst_ref[...] = a_d.reshape(1, 1, ROW_BLK)
    bs = jnp.max(a_s)
    bd = jnp.max(a_d)

    @pl.when(i == 0)
    def _init():
        mx_s[0] = bs
        mx_s[1] = bd

    @pl.when(i > 0)
    def _acc():
        mx_s[0] = jnp.maximum(mx_s[0], bs)
        mx_s[1] = jnp.maximum(mx_s[1], bd)

    mm = mx_s[0] + mx_s[1]
    mm = jnp.where(mm >= 0.0, mm, mm * NEG_SLOPE)
    m_ref[...] = jnp.full((C,), mm, dtype=jnp.float32)


def _prep(x, W, att_s, att_d):
    return pl.pallas_call(
        _prep_body,
        grid=(GRID,),
        in_specs=[
            pl.BlockSpec((ROW_BLK, C), lambda i: (i, 0)),
            pl.BlockSpec((C, C), lambda i: (0, 0)),
            pl.BlockSpec((1, C), lambda i: (0, 0)),
            pl.BlockSpec((1, C), lambda i: (0, 0)),
        ],
        out_specs=[
            pl.BlockSpec((ROW_BLK, C), lambda i: (i, 0)),
            pl.BlockSpec((1, 1, ROW_BLK), lambda i: (i, 0, 0)),
            pl.BlockSpec((1, 1, ROW_BLK), lambda i: (i, 0, 0)),
            pl.BlockSpec((C,), lambda i: (0,)),
        ],
        out_shape=[
            jax.ShapeDtypeStruct((N, C), jnp.float32),
            jax.ShapeDtypeStruct((GRID, 1, ROW_BLK), jnp.float32),
            jax.ShapeDtypeStruct((GRID, 1, ROW_BLK), jnp.float32),
            jax.ShapeDtypeStruct((C,), jnp.float32),
        ],
        scratch_shapes=[pltpu.SMEM((2,), jnp.float32)],
    )(x, W, att_s, att_d)


def _sc_body(h_hbm, asrc_hbm, adst_hbm, m_hbm, eidx_hbm,
             zr_hbm, zv_hbm,
             outp_hbm, denp_hbm,
             asrc_t, adst_t, m_v, ex_a, ex_b,
             rows_a, rows_b, sd_a, sd_b,
             acc, den_sh, ga, gb, sa, sb):
    cid = lax.axis_index("c")
    sid = lax.axis_index("s")
    cbase = (sid * NC + cid) * NCHUNK

    pltpu.sync_copy(asrc_hbm, asrc_t)
    pltpu.sync_copy(adst_hbm, adst_t)
    pltpu.sync_copy(m_hbm.at[pl.ds(0, 16)], m_v)
    pltpu.sync_copy(zr_hbm, acc.at[pl.ds(sid * RPT, RPT)])

    @pl.when(sid == 0)
    def _zero_den():
        pltpu.sync_copy(zv_hbm, den_sh)

    mv = m_v[...]

    plsc.subcore_barrier()

    def fire(ci, sdbuf, rbuf, gsem):
        pltpu.sync_copy(eidx_hbm.at[cbase + ci], sdbuf)
        pltpu.async_copy(h_hbm.at[sdbuf.at[0]], rbuf, gsem)

    def wait_g(sdbuf, rbuf, gsem):
        pltpu.make_async_copy(h_hbm.at[sdbuf.at[0]], rbuf, gsem).wait()

    def compute(sdbuf, rbuf, exb):
        for g in range(K // 16):
            sv = sdbuf[0, pl.ds(g * 16, 16)]
            dv = sdbuf[1, pl.ds(g * 16, 16)]
            a1 = plsc.load_gather(asrc_t, [sv])
            a2 = plsc.load_gather(adst_t, [dv])
            al = a1 + a2
            al = jnp.where(al >= 0.0, al, al * NEG_SLOPE)
            exv = jnp.exp(al - mv)
            exb[pl.ds(g * 16, 16)] = exv
            for e2 in range(16):
                s = exv[e2]
                row = g * 16 + e2
                for j in range(C // 16):
                    rbuf[row, pl.ds(j * 16, 16)] = (
                        rbuf[row, pl.ds(j * 16, 16)] * s)

    def scat(sdbuf, rbuf, exb, ssem):
        pltpu.async_copy(exb, den_sh.at[sdbuf.at[1]], ssem, add=True)
        pltpu.async_copy(rbuf, acc.at[sdbuf.at[1]], ssem, add=True)

    def wait_s(sdbuf, rbuf, exb, ssem):
        pltpu.make_async_copy(exb, den_sh.at[sdbuf.at[1]], ssem).wait()
        pltpu.make_async_copy(rbuf, acc.at[sdbuf.at[1]], ssem).wait()

    fire(0, sd_a, rows_a, ga)
    wait_g(sd_a, rows_a, ga)
    fire(1, sd_b, rows_b, gb)
    compute(sd_a, rows_a, ex_a)
    scat(sd_a, rows_a, ex_a, sa)

    def pair(j, carry):
        cb = 2 * j + 1
        wait_g(sd_b, rows_b, gb)
        wait_s(sd_a, rows_a, ex_a, sa)
        fire(cb + 1, sd_a, rows_a, ga)
        compute(sd_b, rows_b, ex_b)
        scat(sd_b, rows_b, ex_b, sb)
        wait_g(sd_a, rows_a, ga)
        wait_s(sd_b, rows_b, ex_b, sb)
        fire(jnp.minimum(cb + 2, NCHUNK - 1), sd_b, rows_b, gb)
        compute(sd_a, rows_a, ex_a)
        scat(sd_a, rows_a, ex_a, sa)
        return carry

    lax.fori_loop(0, (NCHUNK - 1) // 2, pair, 0)

    wait_g(sd_b, rows_b, gb)
    wait_s(sd_a, rows_a, ex_a, sa)

    plsc.subcore_barrier()

    pltpu.sync_copy(acc.at[pl.ds(sid * RPT, RPT)],
                    outp_hbm.at[cid, pl.ds(sid * RPT, RPT)])

    @pl.when(sid == 0)
    def _den_out():
        pltpu.sync_copy(den_sh, asrc_t)
        pltpu.sync_copy(asrc_t, denp_hbm.at[cid])


def _edge_pass(h, a_src, a_dst, m, eidx, zr, zv):
    mesh = plsc.VectorSubcoreMesh(core_axis_name="c", subcore_axis_name="s")
    fn = functools.partial(
        pl.kernel,
        mesh=mesh,
        compiler_params=pltpu.CompilerParams(needs_layout_passes=False),
        out_type=[
            jax.ShapeDtypeStruct((NC, NA, C), jnp.float32),
            jax.ShapeDtypeStruct((NC, N), jnp.float32),
        ],
        scratch_types=[
            pltpu.VMEM((N,), jnp.float32),
            pltpu.VMEM((N,), jnp.float32),
            pltpu.VMEM((16,), jnp.float32),
            pltpu.VMEM((K,), jnp.float32),
            pltpu.VMEM((K,), jnp.float32),
            pltpu.VMEM((K, C), jnp.float32),
            pltpu.VMEM((K, C), jnp.float32),
            pltpu.VMEM((2, K), jnp.int32),
            pltpu.VMEM((2, K), jnp.int32),
            pltpu.VMEM_SHARED((NA, C), jnp.float32),
            pltpu.VMEM_SHARED((N,), jnp.float32),
            pltpu.SemaphoreType.DMA,
            pltpu.SemaphoreType.DMA,
            pltpu.SemaphoreType.DMA,
            pltpu.SemaphoreType.DMA,
        ],
    )(_sc_body)
    return fn(h, a_src, a_dst, m, eidx, zr, zv)


def _finish_body(outp_ref, denp_ref, h_ref, asrc_ref, adst_ref, m_ref, b_ref,
                 o_ref):
    s = outp_ref[0] + outp_ref[1]
    d = jnp.sum(denp_ref[:, 0, 0, :], axis=0)
    mm = jnp.max(m_ref[...])
    al = asrc_ref[0, 0] + adst_ref[0, 0]
    al = jnp.where(al >= 0.0, al, al * NEG_SLOPE)
    exl = jnp.exp(al - mm)
    s = s + exl[:, None] * h_ref[...]
    d = d + exl
    o = s / (d + 1e-16)[:, None] + b_ref[...][None, :]
    mx = jnp.max(o, axis=1, keepdims=True)
    lo = o - mx
    o_ref[...] = lo - jnp.log(jnp.sum(jnp.exp(lo), axis=1, keepdims=True))


def _finish(outp, denp, h, a_src, a_dst, m, bias):
    return pl.pallas_call(
        _finish_body,
        grid=(GRID,),
        in_specs=[
            pl.BlockSpec((NC, ROW_BLK, C), lambda i: (0, i, 0)),
            pl.BlockSpec((NC, 1, 1, ROW_BLK), lambda i: (0, i, 0, 0)),
            pl.BlockSpec((ROW_BLK, C), lambda i: (i, 0)),
            pl.BlockSpec((1, 1, ROW_BLK), lambda i: (i, 0, 0)),
            pl.BlockSpec((1, 1, ROW_BLK), lambda i: (i, 0, 0)),
            pl.BlockSpec((C,), lambda i: (0,)),
            pl.BlockSpec((C,), lambda i: (0,)),
        ],
        out_specs=pl.BlockSpec((ROW_BLK, C), lambda i: (i, 0)),
        out_shape=jax.ShapeDtypeStruct((N, C), jnp.float32),
    )(outp, denp, h, a_src, a_dst, m, bias)


def kernel(x, edge_index, W, att_src, att_dst, bias):
    att_s = att_src.reshape(1, C)
    att_d = att_dst.reshape(1, C)
    h, a_src3, a_dst3, m = _prep(x, W, att_s, att_d)
    a_src = a_src3.reshape(N)
    a_dst = a_dst3.reshape(N)
    src = edge_index[0]
    dst = edge_index[1]
    eidx = jnp.stack([src.reshape(E // K, K), dst.reshape(E // K, K)], axis=1)
    zr = jnp.zeros((RPT, C), jnp.float32)
    zv = jnp.zeros((N,), jnp.float32)
    outp, denp = _edge_pass(h, a_src, a_dst, m, eidx, zr, zv)
    denp_r = denp.reshape(NC, GRID, 1, ROW_BLK)
    return _finish(outp[:, :N, :], denp_r, h, a_src3, a_dst3, m, bias)
